# baseline (speedup 1.0000x reference)
import functools

import jax
import jax.numpy as jnp
from jax import lax
from jax.experimental import pallas as pl
from jax.experimental.pallas import tpu as pltpu
from jax.experimental.pallas import tpu_sc as plsc

N = 2048
E = 131072
D = 16
H = 8
NN = N * N
RB = 17
RROWS = 1 << RB
NPASS = (NN // RROWS) // 2
TPB = E // 16
DUMP0 = RROWS
ACC_ROWS = RROWS + 256
CBUF = TPB + 128

BE = 8192


def _proj_body(ei_ref, x_ref, wt_ref, b_ref, proj_ref, flat_ref):
    proj_ref[...] = (
        jnp.dot(x_ref[...], wt_ref[...], preferred_element_type=jnp.float32)
        + b_ref[...]
    )
    flat_ref[...] = ei_ref[0, :] * N + ei_ref[1, :]


_sc_mesh = plsc.VectorSubcoreMesh(core_axis_name="c", subcore_axis_name="s")


@functools.partial(
    pl.kernel,
    out_type=jax.ShapeDtypeStruct((NN, H), jnp.float32),
    mesh=_sc_mesh,
    scratch_types=[
        pltpu.VMEM((TPB,), jnp.int32),
        pltpu.VMEM((CBUF,), jnp.int32),
        pltpu.VMEM((CBUF,), jnp.int32),
        pltpu.VMEM((128, H), jnp.float32),
        pltpu.VMEM((1024, H), jnp.float32),
        pltpu.VMEM_SHARED((ACC_ROWS, H), jnp.float32),
        pltpu.SemaphoreType.DMA,
    ],
    compiler_params=pltpu.CompilerParams(
        use_tc_tiling_on_sc=False, needs_layout_passes=False),
)
def _sc_scatter(flat_hbm, proj_hbm, zeros_hbm, out_hbm,
                idx_v, ceid, cidx, rowbuf, zero_v, acc, sem):
    c = lax.axis_index("c")
    s = lax.axis_index("s")
    base = s * TPB

    pltpu.sync_copy(flat_hbm.at[pl.ds(base, TPB)], idx_v)
    pltpu.sync_copy(zeros_hbm, zero_v)
    dump_vec = jnp.full((16,), DUMP0 + 8 * s, jnp.int32)
    zero_vec = jnp.zeros((16,), jnp.int32)
    lane = jnp.arange(16, dtype=jnp.int32)

    def init_body(v, carry):
        ceid[pl.ds(v * 16, 16)] = zero_vec
        return carry
    lax.fori_loop(0, CBUF // 16, init_body, 0)

    def pass_body(p, carry):
        r = 2 * p + c

        def zbody(q, carry2):
            pltpu.sync_copy(zero_v, acc.at[pl.ds(s * 8192 + q * 1024, 1024), :])
            return carry2
        lax.fori_loop(0, 8, zbody, 0)

        r_vec = jnp.full((16,), r, jnp.int32)

        def cbody(v, cursor):
            vec = plsc.load_gather(
                idx_v, [jnp.full((16,), v * 16, jnp.int32) + lane])
            rid = lax.shift_right_logical(vec, RB)
            m = rid == r_vec
            mi = m.astype(jnp.int32)
            pos = jnp.full((16,), cursor - 1, jnp.int32) + plsc.cumsum(mi)
            evec = jnp.full((16,), base + v * 16, jnp.int32) + lane
            plsc.store_scatter(ceid, [pos], evec, mask=m)
            plsc.store_scatter(
                cidx, [pos], jnp.bitwise_and(vec, RROWS - 1), mask=m)
            return cursor + jnp.sum(mi)
        cursor = lax.fori_loop(0, TPB // 16, cbody, 0)

        cur_vec = jnp.full((16,), cursor, jnp.int32) + lane
        plsc.store_scatter(ceid, [cur_vec], zero_vec)
        plsc.store_scatter(cidx, [cur_vec], dump_vec)

        plsc.subcore_barrier()

        n16 = (cursor + 15) // 16
        n128 = (cursor + 127) // 128

        def gbody(g, carry2):
            pltpu.async_copy(
                proj_hbm.at[ceid.at[pl.ds(g * 128, 128)]], rowbuf, sem).wait()
            tmax = jnp.minimum(8, n16 - g * 8)

            def tbody(t, carry3):
                idxv = plsc.load_gather(
                    cidx,
                    [jnp.full((16,), g * 128 + t * 16, jnp.int32) + lane])
                pltpu.sync_copy(
                    rowbuf.at[pl.ds(t * 16, 16), :], acc.at[idxv], add=True)
                return carry3
            return lax.fori_loop(0, tmax, tbody, carry2)
        lax.fori_loop(0, n128, gbody, 0)

        plsc.subcore_barrier()

        pltpu.sync_copy(
            acc.at[pl.ds(s * 8192, 8192), :],
            out_hbm.at[pl.ds(r * RROWS + s * 8192, 8192), :],
        )
        return carry
    lax.fori_loop(0, NPASS, pass_body, 0)


def kernel(edge_index, edge_attr, num_nodes, W, b):
    del num_nodes
    wt = W.T
    proj, flat = pl.pallas_call(
        _proj_body,
        grid=(E // BE,),
        in_specs=[
            pl.BlockSpec((2, BE), lambda g: (0, g)),
            pl.BlockSpec((BE, D), lambda g: (g, 0)),
            pl.BlockSpec((D, H), lambda g: (0, 0)),
            pl.BlockSpec((1, H), lambda g: (0, 0)),
        ],
        out_specs=[
            pl.BlockSpec((BE, H), lambda g: (g, 0)),
            pl.BlockSpec((BE,), lambda g: (g,)),
        ],
        out_shape=[
            jax.ShapeDtypeStruct((E, H), jnp.float32),
            jax.ShapeDtypeStruct((E,), jnp.int32),
        ],
    )(edge_index.astype(jnp.int32), edge_attr, wt, b.reshape(1, H))

    zeros_src = jnp.zeros((1024, H), jnp.float32)
    out = _sc_scatter(flat, proj, zeros_src)
    return out.reshape(N, N, H)
